# baseline (speedup 1.0000x reference)
import functools

import jax
import jax.numpy as jnp
from jax import lax
from jax.experimental import pallas as pl
from jax.experimental.pallas import tpu as pltpu
from jax.experimental.pallas import tpu_sc as plsc

NN = 33
NCLS = 1000
NQ = 1024
D = 128
K = 100000
SUB = 128
NB = 784
KPAD = NB * SUB
BQ = 128
BK = 2048
CAND = NN * SUB

R = NQ * NN
NWORK = 32
RPW = R // NWORK
CH = 96
NCHUNK = RPW // CH
NBUF = 3


def _dist_kernel(z_ref, kt_ref, d2_ref, bm_ref):
    j = pl.program_id(0)
    z = z_ref[...]
    kt = kt_ref[...]
    qsq = jnp.sum(z * z, axis=1, keepdims=True)
    ksq = jnp.sum(kt * kt, axis=0, keepdims=True)
    dot = jnp.dot(z, kt, preferred_element_type=jnp.float32)
    d2 = qsq - 2.0 * dot + ksq
    col = j * BK + lax.broadcasted_iota(jnp.int32, (NQ, BK), 1)
    d2 = jnp.where(col < K, d2, jnp.inf)
    d2_ref[...] = d2
    mins = [jnp.min(d2[:, s * SUB:(s + 1) * SUB], axis=1, keepdims=True)
            for s in range(BK // SUB)]
    bm_ref[0, 0, :, :] = jnp.concatenate(mins, axis=1)


def _select_kernel(bm_ref, rows_ref, blks_ref):
    bm = bm_ref[...]
    cols = lax.broadcasted_iota(jnp.int32, (NQ, NB), 1)
    qio = lax.broadcasted_iota(jnp.int32, (NQ, 1), 0)
    for t in range(NN):
        m = jnp.min(bm, axis=1, keepdims=True)
        pos = jnp.min(jnp.where(bm == m, cols, NB), axis=1, keepdims=True)
        bm = jnp.where(cols == pos, jnp.inf, bm)
        rows_ref[:, pl.ds(t, 1)] = pos + NB * qio
        blks_ref[:, pl.ds(t, 1)] = pos


MASK31 = 0x7FFFFFFF
BIGS = 1900671690


def _sortable(c):
    ci = lax.bitcast_convert_type(c, jnp.int32)
    return jnp.where(ci >= 0, ci, ci ^ MASK31)


def _thresh_kernel(c_ref, t_ref, tc_ref):
    s = jnp.minimum(jnp.maximum(_sortable(c_ref[...]), -2), BIGS)
    cols = lax.broadcasted_iota(jnp.int32, (BQ, CAND), 1)

    def vbody(_, carry):
        lo, hi = carry
        mid = lo + lax.shift_right_arithmetic(hi - lo, 1)
        cnt = jnp.sum((s <= mid).astype(jnp.int32), axis=1, keepdims=True)
        ge = cnt >= NN
        return jnp.where(ge, lo, mid), jnp.where(ge, mid, hi)

    lo0 = jnp.full((BQ, 1), -3, jnp.int32)
    hi0 = jnp.full((BQ, 1), BIGS, jnp.int32)
    _, t = lax.fori_loop(0, 31, vbody, (lo0, hi0))

    need = NN - jnp.sum((s < t).astype(jnp.int32), axis=1, keepdims=True)
    cm = jnp.where(s == t, cols, CAND)

    def cbody(_, carry):
        lo, hi = carry
        mid = lo + lax.shift_right_arithmetic(hi - lo, 1)
        cnt = jnp.sum((cm <= mid).astype(jnp.int32), axis=1, keepdims=True)
        ge = cnt >= need
        return jnp.where(ge, lo, mid), jnp.where(ge, mid, hi)

    lo0c = jnp.full((BQ, 1), -1, jnp.int32)
    hi0c = jnp.full((BQ, 1), CAND - 1, jnp.int32)
    _, tcol = lax.fori_loop(0, 13, cbody, (lo0c, hi0c))

    t_ref[...] = jnp.broadcast_to(t, (BQ, 16))
    tc_ref[...] = jnp.broadcast_to(tcol, (BQ, 16))


def _sc_gather(d2_table, lab_table, rows, blks):
    mesh = plsc.VectorSubcoreMesh(core_axis_name="c", subcore_axis_name="s")

    @functools.partial(
        pl.kernel,
        mesh=mesh,
        out_type=(
            jax.ShapeDtypeStruct((R, SUB), jnp.float32),
            jax.ShapeDtypeStruct((R, SUB), jnp.int32),
        ),
        scratch_types=[
            [pltpu.VMEM((CH,), jnp.int32) for _ in range(NBUF)],
            [pltpu.VMEM((CH,), jnp.int32) for _ in range(NBUF)],
            [pltpu.VMEM((CH, SUB), jnp.float32) for _ in range(NBUF)],
            [pltpu.VMEM((CH, SUB), jnp.int32) for _ in range(NBUF)],
            [pltpu.SemaphoreType.DMA for _ in range(6 * NBUF)],
        ],
    )
    def gather(d2_hbm, lab_hbm, rows_hbm, blks_hbm, cand_hbm, clab_hbm,
               ridx_v, bidx_v, rows_v, labs_v, sems):
        wid = lax.axis_index("s") * 2 + lax.axis_index("c")
        base = wid * RPW
        gcp = [None] * NBUF
        ocp = [None] * NBUF

        def fire(ch):
            b = ch % NBUF
            off = base + ch * CH
            i1 = pltpu.async_copy(rows_hbm.at[pl.ds(off, CH)], ridx_v[b],
                                  sems[6 * b])
            i2 = pltpu.async_copy(blks_hbm.at[pl.ds(off, CH)], bidx_v[b],
                                  sems[6 * b + 1])
            i1.wait()
            i2.wait()
            g1 = pltpu.async_copy(d2_hbm.at[ridx_v[b]], rows_v[b],
                                  sems[6 * b + 2])
            g2 = pltpu.async_copy(lab_hbm.at[bidx_v[b]], labs_v[b],
                                  sems[6 * b + 3])
            gcp[b] = (g1, g2)

        for ch in range(min(NBUF, NCHUNK)):
            fire(ch)
        for ch in range(NCHUNK):
            b = ch % NBUF
            off = base + ch * CH
            gcp[b][0].wait()
            gcp[b][1].wait()
            o1 = pltpu.async_copy(rows_v[b], cand_hbm.at[pl.ds(off, CH)],
                                  sems[6 * b + 4])
            o2 = pltpu.async_copy(labs_v[b], clab_hbm.at[pl.ds(off, CH)],
                                  sems[6 * b + 5])
            ocp[b] = (o1, o2)
            if ch + NBUF < NCHUNK:
                ocp[b][0].wait()
                ocp[b][1].wait()
                fire(ch + NBUF)
        for b in range(min(NBUF, NCHUNK)):
            ocp[b][0].wait()
            ocp[b][1].wait()

    return gather(d2_table, lab_table, rows, blks)


QPW = NQ // NWORK
NCLS_PAD = 1008


def _sc_vote(cand, clab, trep, tcrep):
    mesh = plsc.VectorSubcoreMesh(core_axis_name="c", subcore_axis_name="s")

    @functools.partial(
        pl.kernel,
        mesh=mesh,
        out_type=jax.ShapeDtypeStruct((NQ, NCLS), jnp.float32),
        compiler_params=pltpu.CompilerParams(
            needs_layout_passes=False, use_tc_tiling_on_sc=False),
        scratch_types=[
            pltpu.VMEM((CAND,), jnp.float32),
            pltpu.VMEM((CAND,), jnp.int32),
            pltpu.VMEM((16,), jnp.int32),
            pltpu.VMEM((16,), jnp.int32),
            pltpu.VMEM((NCLS_PAD,), jnp.float32),
        ],
    )
    def vote(cand_hbm, clab_hbm, t_hbm, tc_hbm, probs_hbm,
             cand_v, clab_v, t_v, tc_v, counts_v):
        wid = lax.axis_index("s") * 2 + lax.axis_index("c")
        lanes = lax.broadcasted_iota(jnp.int32, (16,), 0)
        ones = jnp.ones((16,), jnp.float32)
        zeros = jnp.zeros((16,), jnp.float32)

        def qbody(qi, _):
            q = wid * QPW + qi
            pltpu.sync_copy(cand_hbm.at[q], cand_v)
            pltpu.sync_copy(clab_hbm.at[q], clab_v)
            pltpu.sync_copy(t_hbm.at[q], t_v)
            pltpu.sync_copy(tc_hbm.at[q], tc_v)
            tv = t_v[...]
            tcv = tc_v[...]

            def zbody(k, _):
                counts_v[pl.ds(k * 16, 16)] = zeros
                return 0

            lax.fori_loop(0, NCLS_PAD // 16, zbody, 0)

            def cbody(i, _):
                cs = _sortable(cand_v[pl.ds(i * 16, 16)])
                lv = clab_v[pl.ds(i * 16, 16)]
                col = lanes + i * 16
                sel = (cs < tv) | ((cs == tv) & (col <= tcv))
                plsc.addupdate_scatter(counts_v, [lv], ones, mask=sel)
                return 0

            lax.fori_loop(0, CAND // 16, cbody, 0)

            def dbody(k, _):
                sl = pl.ds(k * 16, 16)
                counts_v[sl] = counts_v[sl] / 33.0
                return 0

            lax.fori_loop(0, NCLS_PAD // 16, dbody, 0)
            pltpu.sync_copy(counts_v.at[pl.ds(0, NCLS)], probs_hbm.at[q])
            return 0

        lax.fori_loop(0, QPW, qbody, 0)

    return vote(cand, clab, trep, tcrep)


def kernel(Z_image, keys, labels):
    kt = jnp.pad(keys, ((0, KPAD - K), (0, 0))).T
    lab_table = jnp.pad(labels, (0, KPAD - K)).reshape(NB, SUB)

    d2, bm3 = pl.pallas_call(
        _dist_kernel,
        grid=(KPAD // BK,),
        in_specs=[
            pl.BlockSpec((NQ, D), lambda j: (0, 0)),
            pl.BlockSpec((D, BK), lambda j: (0, j)),
        ],
        out_specs=[
            pl.BlockSpec((NQ, BK), lambda j: (0, j)),
            pl.BlockSpec((1, 1, NQ, BK // SUB), lambda j: (0, j, 0, 0)),
        ],
        out_shape=[
            jax.ShapeDtypeStruct((NQ, KPAD), jnp.float32),
            jax.ShapeDtypeStruct(
                (1, KPAD // BK, NQ, BK // SUB), jnp.float32),
        ],
    )(Z_image, kt)
    bm = bm3.reshape(KPAD // BK, NQ, BK // SUB).transpose(1, 0, 2).reshape(NQ, NB)

    rows, blks = pl.pallas_call(
        _select_kernel,
        in_specs=[pl.BlockSpec((NQ, NB), lambda: (0, 0))],
        out_specs=[
            pl.BlockSpec((NQ, NN), lambda: (0, 0)),
            pl.BlockSpec((NQ, NN), lambda: (0, 0)),
        ],
        out_shape=[
            jax.ShapeDtypeStruct((NQ, NN), jnp.int32),
            jax.ShapeDtypeStruct((NQ, NN), jnp.int32),
        ],
    )(bm)

    cand, clab = _sc_gather(
        d2.reshape(NQ * NB, SUB), lab_table,
        rows.reshape(R), blks.reshape(R))

    cand2 = cand.reshape(NQ, CAND)
    clab2 = clab.reshape(NQ, CAND)

    trep, tcrep = pl.pallas_call(
        _thresh_kernel,
        grid=(NQ // BQ,),
        in_specs=[pl.BlockSpec((BQ, CAND), lambda i: (i, 0))],
        out_specs=[
            pl.BlockSpec((BQ, 16), lambda i: (i, 0)),
            pl.BlockSpec((BQ, 16), lambda i: (i, 0)),
        ],
        out_shape=[
            jax.ShapeDtypeStruct((NQ, 16), jnp.int32),
            jax.ShapeDtypeStruct((NQ, 16), jnp.int32),
        ],
    )(cand2)

    return _sc_vote(cand2, clab2, trep, tcrep)


# speedup vs baseline: 1.0890x; 1.0890x over previous
import functools

import jax
import jax.numpy as jnp
from jax import lax
from jax.experimental import pallas as pl
from jax.experimental.pallas import tpu as pltpu
from jax.experimental.pallas import tpu_sc as plsc

NN = 33
NCLS = 1000
NQ = 1024
D = 128
K = 100000
SUB = 128
NB = 784
KPAD = NB * SUB
BQ = 128
BK = 2048
CAND = NN * SUB

R = NQ * NN
NWORK = 32
RPW = R // NWORK
CH = 96
NCHUNK = RPW // CH
NBUF = 3


def _dist_kernel(z_ref, kt_ref, d2_ref, bm_ref):
    j = pl.program_id(0)
    z = z_ref[...]
    kt = kt_ref[...]
    qsq = jnp.sum(z * z, axis=1, keepdims=True)
    ksq = jnp.sum(kt * kt, axis=0, keepdims=True)
    dot = jnp.dot(z, kt, preferred_element_type=jnp.float32)
    d2 = qsq - 2.0 * dot + ksq
    col = j * BK + lax.broadcasted_iota(jnp.int32, (NQ, BK), 1)
    d2 = jnp.where(col < K, d2, jnp.inf)
    d2_ref[...] = d2
    mins = [jnp.min(d2[:, s * SUB:(s + 1) * SUB], axis=1, keepdims=True)
            for s in range(BK // SUB)]
    bm_ref[0, 0, :, :] = jnp.concatenate(mins, axis=1)


def _select_kernel(bm_ref, rows_ref, blks_ref):
    bm = bm_ref[...]
    cols = lax.broadcasted_iota(jnp.int32, (NQ, NB), 1)
    qio = lax.broadcasted_iota(jnp.int32, (NQ, 1), 0)
    for t in range(NN):
        m = jnp.min(bm, axis=1, keepdims=True)
        pos = jnp.min(jnp.where(bm == m, cols, NB), axis=1, keepdims=True)
        bm = jnp.where(cols == pos, jnp.inf, bm)
        rows_ref[:, pl.ds(t, 1)] = pos + NB * qio
        blks_ref[:, pl.ds(t, 1)] = pos


MASK31 = 0x7FFFFFFF
BIGS = 1900671690


def _sortable(c):
    ci = lax.bitcast_convert_type(c, jnp.int32)
    return jnp.where(ci >= 0, ci, ci ^ MASK31)


def _thresh_kernel(c_ref, t_ref, tc_ref):
    s = jnp.minimum(jnp.maximum(_sortable(c_ref[...]), -2), BIGS)
    cols = lax.broadcasted_iota(jnp.int32, (BQ, CAND), 1)

    def vbody(_, carry):
        lo, hi = carry
        mid = lo + lax.shift_right_arithmetic(hi - lo, 1)
        cnt = jnp.sum((s <= mid).astype(jnp.int32), axis=1, keepdims=True)
        ge = cnt >= NN
        return jnp.where(ge, lo, mid), jnp.where(ge, mid, hi)

    lo0 = jnp.full((BQ, 1), -3, jnp.int32)
    hi0 = jnp.full((BQ, 1), BIGS, jnp.int32)
    _, t = lax.fori_loop(0, 31, vbody, (lo0, hi0))

    need = NN - jnp.sum((s < t).astype(jnp.int32), axis=1, keepdims=True)
    cm = jnp.where(s == t, cols, CAND)

    def cbody(_, carry):
        lo, hi = carry
        mid = lo + lax.shift_right_arithmetic(hi - lo, 1)
        cnt = jnp.sum((cm <= mid).astype(jnp.int32), axis=1, keepdims=True)
        ge = cnt >= need
        return jnp.where(ge, lo, mid), jnp.where(ge, mid, hi)

    lo0c = jnp.full((BQ, 1), -1, jnp.int32)
    hi0c = jnp.full((BQ, 1), CAND - 1, jnp.int32)
    _, tcol = lax.fori_loop(0, 13, cbody, (lo0c, hi0c))

    t_ref[...] = jnp.broadcast_to(t, (BQ, 16))
    tc_ref[...] = jnp.broadcast_to(tcol, (BQ, 16))


def _sc_gather(d2_table, lab_table, rows, blks):
    mesh = plsc.VectorSubcoreMesh(core_axis_name="c", subcore_axis_name="s")

    @functools.partial(
        pl.kernel,
        mesh=mesh,
        out_type=(
            jax.ShapeDtypeStruct((R, SUB), jnp.float32),
            jax.ShapeDtypeStruct((R, SUB), jnp.int32),
        ),
        scratch_types=[
            [pltpu.VMEM((CH,), jnp.int32) for _ in range(NBUF)],
            [pltpu.VMEM((CH,), jnp.int32) for _ in range(NBUF)],
            [pltpu.VMEM((CH, SUB), jnp.float32) for _ in range(NBUF)],
            [pltpu.VMEM((CH, SUB), jnp.int32) for _ in range(NBUF)],
            [pltpu.SemaphoreType.DMA for _ in range(6 * NBUF)],
        ],
    )
    def gather(d2_hbm, lab_hbm, rows_hbm, blks_hbm, cand_hbm, clab_hbm,
               ridx_v, bidx_v, rows_v, labs_v, sems):
        wid = lax.axis_index("s") * 2 + lax.axis_index("c")
        base = wid * RPW
        gcp = [None] * NBUF
        ocp = [None] * NBUF

        def fire(ch):
            b = ch % NBUF
            off = base + ch * CH
            i1 = pltpu.async_copy(rows_hbm.at[pl.ds(off, CH)], ridx_v[b],
                                  sems[6 * b])
            i2 = pltpu.async_copy(blks_hbm.at[pl.ds(off, CH)], bidx_v[b],
                                  sems[6 * b + 1])
            i1.wait()
            i2.wait()
            g1 = pltpu.async_copy(d2_hbm.at[ridx_v[b]], rows_v[b],
                                  sems[6 * b + 2])
            g2 = pltpu.async_copy(lab_hbm.at[bidx_v[b]], labs_v[b],
                                  sems[6 * b + 3])
            gcp[b] = (g1, g2)

        for ch in range(min(NBUF, NCHUNK)):
            fire(ch)
        for ch in range(NCHUNK):
            b = ch % NBUF
            off = base + ch * CH
            gcp[b][0].wait()
            gcp[b][1].wait()
            o1 = pltpu.async_copy(rows_v[b], cand_hbm.at[pl.ds(off, CH)],
                                  sems[6 * b + 4])
            o2 = pltpu.async_copy(labs_v[b], clab_hbm.at[pl.ds(off, CH)],
                                  sems[6 * b + 5])
            ocp[b] = (o1, o2)
            if ch + NBUF < NCHUNK:
                ocp[b][0].wait()
                ocp[b][1].wait()
                fire(ch + NBUF)
        for b in range(min(NBUF, NCHUNK)):
            ocp[b][0].wait()
            ocp[b][1].wait()

    return gather(d2_table, lab_table, rows, blks)


QPW = NQ // NWORK
NCLS_PAD = 1008


VB = 4
NVR = QPW // VB


def _sc_vote(cand_f, clab_f, trep_f, tcrep_f):
    mesh = plsc.VectorSubcoreMesh(core_axis_name="c", subcore_axis_name="s")

    @functools.partial(
        pl.kernel,
        mesh=mesh,
        out_type=jax.ShapeDtypeStruct((NQ * NCLS,), jnp.float32),
        compiler_params=pltpu.CompilerParams(
            needs_layout_passes=False, use_tc_tiling_on_sc=False),
        scratch_types=[
            [pltpu.VMEM((VB * CAND,), jnp.float32) for _ in range(2)],
            [pltpu.VMEM((VB * CAND,), jnp.int32) for _ in range(2)],
            pltpu.VMEM((QPW * 16,), jnp.int32),
            pltpu.VMEM((QPW * 16,), jnp.int32),
            [pltpu.VMEM((VB * NCLS,), jnp.float32) for _ in range(2)],
            [pltpu.SemaphoreType.DMA for _ in range(6)],
        ],
    )
    def vote(cand_hbm, clab_hbm, t_hbm, tc_hbm, probs_hbm,
             cbuf, lbuf, tall_v, tcall_v, counts, sems):
        wid = lax.axis_index("s") * 2 + lax.axis_index("c")
        base_q = wid * QPW
        lanes = lax.broadcasted_iota(jnp.int32, (16,), 0)
        ones = jnp.ones((16,), jnp.float32)
        zeros = jnp.zeros((16,), jnp.float32)

        pltpu.sync_copy(t_hbm.at[pl.ds(base_q * 16, QPW * 16)], tall_v)
        pltpu.sync_copy(tc_hbm.at[pl.ds(base_q * 16, QPW * 16)], tcall_v)

        lc = [None, None]
        ll = [None, None]
        ocp = [None, None]

        def fire(r):
            b = r % 2
            off = (base_q + r * VB) * CAND
            lc[b] = pltpu.async_copy(cand_hbm.at[pl.ds(off, VB * CAND)],
                                     cbuf[b], sems[2 * b])
            ll[b] = pltpu.async_copy(clab_hbm.at[pl.ds(off, VB * CAND)],
                                     lbuf[b], sems[2 * b + 1])

        fire(0)
        fire(1)
        for r in range(NVR):
            b = r % 2
            lc[b].wait()
            ll[b].wait()
            if r >= 2:
                ocp[b].wait()

            def zbody(k, _):
                counts[b][pl.ds(k * 16, 16)] = zeros
                return 0

            lax.fori_loop(0, VB * NCLS // 16, zbody, 0)

            for qi in range(VB):
                qrow = r * VB + qi
                tv = tall_v[pl.ds(qrow * 16, 16)]
                tcv = tcall_v[pl.ds(qrow * 16, 16)]

                def cbody(i, _):
                    for u in range(8):
                        off0 = i * 128 + u * 16
                        cs = _sortable(cbuf[b][pl.ds(qi * CAND + off0, 16)])
                        lv = lbuf[b][pl.ds(qi * CAND + off0, 16)]
                        col = lanes + off0
                        sel = (cs < tv) | ((cs == tv) & (col <= tcv))
                        plsc.addupdate_scatter(
                            counts[b], [lv + qi * NCLS], ones, mask=sel)
                    return 0

                lax.fori_loop(0, CAND // 128, cbody, 0)

            def dbody(k, _):
                sl = pl.ds(k * 16, 16)
                counts[b][sl] = counts[b][sl] / 33.0
                return 0

            lax.fori_loop(0, VB * NCLS // 16, dbody, 0)
            ocp[b] = pltpu.async_copy(
                counts[b],
                probs_hbm.at[pl.ds((base_q + r * VB) * NCLS, VB * NCLS)],
                sems[4 + b])
            if r + 2 < NVR:
                fire(r + 2)
        ocp[0].wait()
        ocp[1].wait()

    return vote(cand_f, clab_f, trep_f, tcrep_f)


def kernel(Z_image, keys, labels):
    kt = jnp.pad(keys, ((0, KPAD - K), (0, 0))).T
    lab_table = jnp.pad(labels, (0, KPAD - K)).reshape(NB, SUB)

    d2, bm3 = pl.pallas_call(
        _dist_kernel,
        grid=(KPAD // BK,),
        in_specs=[
            pl.BlockSpec((NQ, D), lambda j: (0, 0)),
            pl.BlockSpec((D, BK), lambda j: (0, j)),
        ],
        out_specs=[
            pl.BlockSpec((NQ, BK), lambda j: (0, j)),
            pl.BlockSpec((1, 1, NQ, BK // SUB), lambda j: (0, j, 0, 0)),
        ],
        out_shape=[
            jax.ShapeDtypeStruct((NQ, KPAD), jnp.float32),
            jax.ShapeDtypeStruct(
                (1, KPAD // BK, NQ, BK // SUB), jnp.float32),
        ],
    )(Z_image, kt)
    bm = bm3.reshape(KPAD // BK, NQ, BK // SUB).transpose(1, 0, 2).reshape(NQ, NB)

    rows, blks = pl.pallas_call(
        _select_kernel,
        in_specs=[pl.BlockSpec((NQ, NB), lambda: (0, 0))],
        out_specs=[
            pl.BlockSpec((NQ, NN), lambda: (0, 0)),
            pl.BlockSpec((NQ, NN), lambda: (0, 0)),
        ],
        out_shape=[
            jax.ShapeDtypeStruct((NQ, NN), jnp.int32),
            jax.ShapeDtypeStruct((NQ, NN), jnp.int32),
        ],
    )(bm)

    cand, clab = _sc_gather(
        d2.reshape(NQ * NB, SUB), lab_table,
        rows.reshape(R), blks.reshape(R))

    cand2 = cand.reshape(NQ, CAND)
    clab2 = clab.reshape(NQ, CAND)

    trep, tcrep = pl.pallas_call(
        _thresh_kernel,
        grid=(NQ // BQ,),
        in_specs=[pl.BlockSpec((BQ, CAND), lambda i: (i, 0))],
        out_specs=[
            pl.BlockSpec((BQ, 16), lambda i: (i, 0)),
            pl.BlockSpec((BQ, 16), lambda i: (i, 0)),
        ],
        out_shape=[
            jax.ShapeDtypeStruct((NQ, 16), jnp.int32),
            jax.ShapeDtypeStruct((NQ, 16), jnp.int32),
        ],
    )(cand2)

    probs_flat = _sc_vote(
        cand.reshape(R * SUB), clab.reshape(R * SUB),
        trep.reshape(NQ * 16), tcrep.reshape(NQ * 16))
    return probs_flat.reshape(NQ, NCLS)


# speedup vs baseline: 1.0987x; 1.0090x over previous
import functools

import jax
import jax.numpy as jnp
from jax import lax
from jax.experimental import pallas as pl
from jax.experimental.pallas import tpu as pltpu
from jax.experimental.pallas import tpu_sc as plsc

NN = 33
NCLS = 1000
NQ = 1024
D = 128
K = 100000
SUB = 128
NB = 784
KPAD = NB * SUB
BQ = 128
BK = 2048
CAND = NN * SUB

R = NQ * NN
NWORK = 32
RPW = R // NWORK
CH = 96
NCHUNK = RPW // CH
NBUF = 3


def _dist_kernel(z_ref, kt_ref, d2_ref, bm_ref):
    j = pl.program_id(0)
    z = z_ref[...]
    kt = kt_ref[...]
    qsq = jnp.sum(z * z, axis=1, keepdims=True)
    ksq = jnp.sum(kt * kt, axis=0, keepdims=True)
    dot = jnp.dot(z, kt, preferred_element_type=jnp.float32)
    d2 = qsq - 2.0 * dot + ksq
    col = j * BK + lax.broadcasted_iota(jnp.int32, (NQ, BK), 1)
    d2 = jnp.where(col < K, d2, jnp.inf)
    d2_ref[...] = d2
    mins = [jnp.min(d2[:, s * SUB:(s + 1) * SUB], axis=1, keepdims=True)
            for s in range(BK // SUB)]
    bm_ref[0, 0, :, :] = jnp.concatenate(mins, axis=1)


def _select_kernel(bm_ref, rows_ref, blks_ref, bmv_ref):
    bm = bm_ref[...]
    cols = lax.broadcasted_iota(jnp.int32, (NQ, NB), 1)
    qio = lax.broadcasted_iota(jnp.int32, (NQ, 1), 0)
    for t in range(NN):
        m = jnp.min(bm, axis=1, keepdims=True)
        pos = jnp.min(jnp.where(bm == m, cols, NB), axis=1, keepdims=True)
        bm = jnp.where(cols == pos, jnp.inf, bm)
        rows_ref[:, pl.ds(t, 1)] = pos + NB * qio
        blks_ref[:, pl.ds(t, 1)] = pos
        bmv_ref[:, pl.ds(t, 1)] = m


MASK31 = 0x7FFFFFFF
BIGS = 1900671690


def _sortable(c):
    ci = lax.bitcast_convert_type(c, jnp.int32)
    return jnp.where(ci >= 0, ci, ci ^ MASK31)


def _thresh_kernel(c_ref, bmv_ref, t_ref, tc_ref):
    s = jnp.minimum(jnp.maximum(_sortable(c_ref[...]), -2), BIGS)
    cols = lax.broadcasted_iota(jnp.int32, (BQ, CAND), 1)

    def vcond(carry):
        lo, hi = carry
        return jnp.max(hi - lo) > 1

    def vbody(carry):
        lo, hi = carry
        mid = lo + lax.shift_right_arithmetic(hi - lo, 1)
        cnt = jnp.sum((s <= mid).astype(jnp.int32), axis=1, keepdims=True)
        ge = cnt >= NN
        return jnp.where(ge, lo, mid), jnp.where(ge, mid, hi)

    lo0 = jnp.maximum(_sortable(bmv_ref[:, 0:1]) - 1, -3)
    hi0 = jnp.minimum(jnp.maximum(_sortable(bmv_ref[:, NN - 1:NN]), -2), BIGS)
    _, t = lax.while_loop(vcond, vbody, (lo0, hi0))

    need = NN - jnp.sum((s < t).astype(jnp.int32), axis=1, keepdims=True)
    cm = jnp.where(s == t, cols, CAND)
    cmin = jnp.min(cm, axis=1, keepdims=True)

    def ccond(carry):
        lo, hi = carry
        return jnp.max(jnp.where(need > 1, hi - lo, 0)) > 1

    def cbody(carry):
        lo, hi = carry
        mid = lo + lax.shift_right_arithmetic(hi - lo, 1)
        cnt = jnp.sum((cm <= mid).astype(jnp.int32), axis=1, keepdims=True)
        ge = cnt >= need
        return jnp.where(ge, lo, mid), jnp.where(ge, mid, hi)

    lo0c = jnp.full((BQ, 1), -1, jnp.int32)
    hi0c = jnp.full((BQ, 1), CAND - 1, jnp.int32)
    _, tcol = lax.while_loop(ccond, cbody, (lo0c, hi0c))
    tcol = jnp.where(need > 1, tcol, cmin)

    t_ref[...] = jnp.broadcast_to(t, (BQ, 16))
    tc_ref[...] = jnp.broadcast_to(tcol, (BQ, 16))


def _sc_gather(d2_table, lab_table, rows, blks):
    mesh = plsc.VectorSubcoreMesh(core_axis_name="c", subcore_axis_name="s")

    @functools.partial(
        pl.kernel,
        mesh=mesh,
        out_type=(
            jax.ShapeDtypeStruct((R, SUB), jnp.float32),
            jax.ShapeDtypeStruct((R, SUB), jnp.int32),
        ),
        scratch_types=[
            [pltpu.VMEM((CH,), jnp.int32) for _ in range(NBUF)],
            [pltpu.VMEM((CH,), jnp.int32) for _ in range(NBUF)],
            [pltpu.VMEM((CH, SUB), jnp.float32) for _ in range(NBUF)],
            [pltpu.VMEM((CH, SUB), jnp.int32) for _ in range(NBUF)],
            [pltpu.SemaphoreType.DMA for _ in range(6 * NBUF)],
        ],
    )
    def gather(d2_hbm, lab_hbm, rows_hbm, blks_hbm, cand_hbm, clab_hbm,
               ridx_v, bidx_v, rows_v, labs_v, sems):
        wid = lax.axis_index("s") * 2 + lax.axis_index("c")
        base = wid * RPW
        gcp = [None] * NBUF
        ocp = [None] * NBUF

        def fire(ch):
            b = ch % NBUF
            off = base + ch * CH
            i1 = pltpu.async_copy(rows_hbm.at[pl.ds(off, CH)], ridx_v[b],
                                  sems[6 * b])
            i2 = pltpu.async_copy(blks_hbm.at[pl.ds(off, CH)], bidx_v[b],
                                  sems[6 * b + 1])
            i1.wait()
            i2.wait()
            g1 = pltpu.async_copy(d2_hbm.at[ridx_v[b]], rows_v[b],
                                  sems[6 * b + 2])
            g2 = pltpu.async_copy(lab_hbm.at[bidx_v[b]], labs_v[b],
                                  sems[6 * b + 3])
            gcp[b] = (g1, g2)

        for ch in range(min(NBUF, NCHUNK)):
            fire(ch)
        for ch in range(NCHUNK):
            b = ch % NBUF
            off = base + ch * CH
            gcp[b][0].wait()
            gcp[b][1].wait()
            o1 = pltpu.async_copy(rows_v[b], cand_hbm.at[pl.ds(off, CH)],
                                  sems[6 * b + 4])
            o2 = pltpu.async_copy(labs_v[b], clab_hbm.at[pl.ds(off, CH)],
                                  sems[6 * b + 5])
            ocp[b] = (o1, o2)
            if ch + NBUF < NCHUNK:
                ocp[b][0].wait()
                ocp[b][1].wait()
                fire(ch + NBUF)
        for b in range(min(NBUF, NCHUNK)):
            ocp[b][0].wait()
            ocp[b][1].wait()

    return gather(d2_table, lab_table, rows, blks)


QPW = NQ // NWORK
NCLS_PAD = 1008


VB = 4
NVR = QPW // VB


def _sc_vote(cand_f, clab_f, trep_f, tcrep_f):
    mesh = plsc.VectorSubcoreMesh(core_axis_name="c", subcore_axis_name="s")

    @functools.partial(
        pl.kernel,
        mesh=mesh,
        out_type=jax.ShapeDtypeStruct((NQ * NCLS,), jnp.float32),
        compiler_params=pltpu.CompilerParams(
            needs_layout_passes=False, use_tc_tiling_on_sc=False),
        scratch_types=[
            [pltpu.VMEM((VB * CAND,), jnp.float32) for _ in range(2)],
            [pltpu.VMEM((VB * CAND,), jnp.int32) for _ in range(2)],
            pltpu.VMEM((QPW * 16,), jnp.int32),
            pltpu.VMEM((QPW * 16,), jnp.int32),
            [pltpu.VMEM((VB * NCLS,), jnp.float32) for _ in range(2)],
            [pltpu.SemaphoreType.DMA for _ in range(6)],
        ],
    )
    def vote(cand_hbm, clab_hbm, t_hbm, tc_hbm, probs_hbm,
             cbuf, lbuf, tall_v, tcall_v, counts, sems):
        wid = lax.axis_index("s") * 2 + lax.axis_index("c")
        base_q = wid * QPW
        lanes = lax.broadcasted_iota(jnp.int32, (16,), 0)
        ones = jnp.ones((16,), jnp.float32)
        zeros = jnp.zeros((16,), jnp.float32)

        pltpu.sync_copy(t_hbm.at[pl.ds(base_q * 16, QPW * 16)], tall_v)
        pltpu.sync_copy(tc_hbm.at[pl.ds(base_q * 16, QPW * 16)], tcall_v)

        lc = [None, None]
        ll = [None, None]
        ocp = [None, None]

        def fire(r):
            b = r % 2
            off = (base_q + r * VB) * CAND
            lc[b] = pltpu.async_copy(cand_hbm.at[pl.ds(off, VB * CAND)],
                                     cbuf[b], sems[2 * b])
            ll[b] = pltpu.async_copy(clab_hbm.at[pl.ds(off, VB * CAND)],
                                     lbuf[b], sems[2 * b + 1])

        fire(0)
        fire(1)
        for r in range(NVR):
            b = r % 2
            lc[b].wait()
            ll[b].wait()
            if r >= 2:
                ocp[b].wait()

            def zbody(k, _):
                counts[b][pl.ds(k * 16, 16)] = zeros
                return 0

            lax.fori_loop(0, VB * NCLS // 16, zbody, 0)

            for qi in range(VB):
                qrow = r * VB + qi
                tv = tall_v[pl.ds(qrow * 16, 16)]
                tcv = tcall_v[pl.ds(qrow * 16, 16)]

                def cbody(i, _):
                    for u in range(8):
                        off0 = i * 128 + u * 16
                        cs = _sortable(cbuf[b][pl.ds(qi * CAND + off0, 16)])
                        lv = lbuf[b][pl.ds(qi * CAND + off0, 16)]
                        col = lanes + off0
                        sel = (cs < tv) | ((cs == tv) & (col <= tcv))
                        plsc.addupdate_scatter(
                            counts[b], [lv + qi * NCLS], ones, mask=sel)
                    return 0

                lax.fori_loop(0, CAND // 128, cbody, 0)

            def dbody(k, _):
                sl = pl.ds(k * 16, 16)
                counts[b][sl] = counts[b][sl] / 33.0
                return 0

            lax.fori_loop(0, VB * NCLS // 16, dbody, 0)
            ocp[b] = pltpu.async_copy(
                counts[b],
                probs_hbm.at[pl.ds((base_q + r * VB) * NCLS, VB * NCLS)],
                sems[4 + b])
            if r + 2 < NVR:
                fire(r + 2)
        ocp[0].wait()
        ocp[1].wait()

    return vote(cand_f, clab_f, trep_f, tcrep_f)


def kernel(Z_image, keys, labels):
    kt = jnp.pad(keys, ((0, KPAD - K), (0, 0))).T
    lab_table = jnp.pad(labels, (0, KPAD - K)).reshape(NB, SUB)

    d2, bm3 = pl.pallas_call(
        _dist_kernel,
        grid=(KPAD // BK,),
        in_specs=[
            pl.BlockSpec((NQ, D), lambda j: (0, 0)),
            pl.BlockSpec((D, BK), lambda j: (0, j)),
        ],
        out_specs=[
            pl.BlockSpec((NQ, BK), lambda j: (0, j)),
            pl.BlockSpec((1, 1, NQ, BK // SUB), lambda j: (0, j, 0, 0)),
        ],
        out_shape=[
            jax.ShapeDtypeStruct((NQ, KPAD), jnp.float32),
            jax.ShapeDtypeStruct(
                (1, KPAD // BK, NQ, BK // SUB), jnp.float32),
        ],
    )(Z_image, kt)
    bm = bm3.reshape(KPAD // BK, NQ, BK // SUB).transpose(1, 0, 2).reshape(NQ, NB)

    rows, blks, bmv = pl.pallas_call(
        _select_kernel,
        in_specs=[pl.BlockSpec((NQ, NB), lambda: (0, 0))],
        out_specs=[
            pl.BlockSpec((NQ, NN), lambda: (0, 0)),
            pl.BlockSpec((NQ, NN), lambda: (0, 0)),
            pl.BlockSpec((NQ, NN), lambda: (0, 0)),
        ],
        out_shape=[
            jax.ShapeDtypeStruct((NQ, NN), jnp.int32),
            jax.ShapeDtypeStruct((NQ, NN), jnp.int32),
            jax.ShapeDtypeStruct((NQ, NN), jnp.float32),
        ],
    )(bm)

    cand, clab = _sc_gather(
        d2.reshape(NQ * NB, SUB), lab_table,
        rows.reshape(R), blks.reshape(R))

    cand2 = cand.reshape(NQ, CAND)
    clab2 = clab.reshape(NQ, CAND)

    trep, tcrep = pl.pallas_call(
        _thresh_kernel,
        grid=(NQ // BQ,),
        in_specs=[
            pl.BlockSpec((BQ, CAND), lambda i: (i, 0)),
            pl.BlockSpec((BQ, NN), lambda i: (i, 0)),
        ],
        out_specs=[
            pl.BlockSpec((BQ, 16), lambda i: (i, 0)),
            pl.BlockSpec((BQ, 16), lambda i: (i, 0)),
        ],
        out_shape=[
            jax.ShapeDtypeStruct((NQ, 16), jnp.int32),
            jax.ShapeDtypeStruct((NQ, 16), jnp.int32),
        ],
    )(cand2, bmv)

    probs_flat = _sc_vote(
        cand.reshape(R * SUB), clab.reshape(R * SUB),
        trep.reshape(NQ * 16), tcrep.reshape(NQ * 16))
    return probs_flat.reshape(NQ, NCLS)


# speedup vs baseline: 1.1058x; 1.0065x over previous
import functools

import jax
import jax.numpy as jnp
from jax import lax
from jax.experimental import pallas as pl
from jax.experimental.pallas import tpu as pltpu
from jax.experimental.pallas import tpu_sc as plsc

NN = 33
NCLS = 1000
NQ = 1024
D = 128
K = 100000
SUB = 128
NB = 784
KPAD = NB * SUB
BQ = 128
BK = 3584
CAND = NN * SUB

R = NQ * NN
NWORK = 32
RPW = R // NWORK
CH = 96
NCHUNK = RPW // CH
NBUF = 3


def _dist_kernel(z_ref, kt_ref, d2_ref, bm_ref):
    j = pl.program_id(0)
    z = z_ref[...]
    kt = kt_ref[...]
    qsq = jnp.sum(z * z, axis=1, keepdims=True)
    ksq = jnp.sum(kt * kt, axis=0, keepdims=True)
    dot = jnp.dot(z, kt, preferred_element_type=jnp.float32)
    d2 = qsq - 2.0 * dot + ksq
    col = j * BK + lax.broadcasted_iota(jnp.int32, (NQ, BK), 1)
    d2 = jnp.where(col < K, d2, jnp.inf)
    d2_ref[...] = d2
    mins = [jnp.min(d2[:, s * SUB:(s + 1) * SUB], axis=1, keepdims=True)
            for s in range(BK // SUB)]
    bm_ref[0, 0, :, :] = jnp.concatenate(mins, axis=1)


def _select_kernel(bm_ref, rows_ref, blks_ref, bmv_ref):
    bm = bm_ref[...]
    cols = lax.broadcasted_iota(jnp.int32, (NQ, NB), 1)
    qio = lax.broadcasted_iota(jnp.int32, (NQ, 1), 0)
    for t in range(NN):
        m = jnp.min(bm, axis=1, keepdims=True)
        pos = jnp.min(jnp.where(bm == m, cols, NB), axis=1, keepdims=True)
        bm = jnp.where(cols == pos, jnp.inf, bm)
        rows_ref[:, pl.ds(t, 1)] = pos + NB * qio
        blks_ref[:, pl.ds(t, 1)] = pos
        bmv_ref[:, pl.ds(t, 1)] = m


MASK31 = 0x7FFFFFFF
BIGS = 1900671690


def _sortable(c):
    ci = lax.bitcast_convert_type(c, jnp.int32)
    return jnp.where(ci >= 0, ci, ci ^ MASK31)


def _thresh_kernel(c_ref, bmv_ref, t_ref, tc_ref):
    s = jnp.minimum(jnp.maximum(_sortable(c_ref[...]), -2), BIGS)
    cols = lax.broadcasted_iota(jnp.int32, (BQ, CAND), 1)

    def vcond(carry):
        lo, hi = carry
        return jnp.max(hi - lo) > 1

    def vbody(carry):
        lo, hi = carry
        mid = lo + lax.shift_right_arithmetic(hi - lo, 1)
        cnt = jnp.sum((s <= mid).astype(jnp.int32), axis=1, keepdims=True)
        ge = cnt >= NN
        return jnp.where(ge, lo, mid), jnp.where(ge, mid, hi)

    lo0 = jnp.maximum(_sortable(bmv_ref[:, 0:1]) - 1, -3)
    hi0 = jnp.minimum(jnp.maximum(_sortable(bmv_ref[:, NN - 1:NN]), -2), BIGS)
    _, t = lax.while_loop(vcond, vbody, (lo0, hi0))

    need = NN - jnp.sum((s < t).astype(jnp.int32), axis=1, keepdims=True)
    cm = jnp.where(s == t, cols, CAND)
    cmin = jnp.min(cm, axis=1, keepdims=True)

    def ccond(carry):
        lo, hi = carry
        return jnp.max(jnp.where(need > 1, hi - lo, 0)) > 1

    def cbody(carry):
        lo, hi = carry
        mid = lo + lax.shift_right_arithmetic(hi - lo, 1)
        cnt = jnp.sum((cm <= mid).astype(jnp.int32), axis=1, keepdims=True)
        ge = cnt >= need
        return jnp.where(ge, lo, mid), jnp.where(ge, mid, hi)

    lo0c = jnp.full((BQ, 1), -1, jnp.int32)
    hi0c = jnp.full((BQ, 1), CAND - 1, jnp.int32)
    _, tcol = lax.while_loop(ccond, cbody, (lo0c, hi0c))
    tcol = jnp.where(need > 1, tcol, cmin)

    t_ref[...] = jnp.broadcast_to(t, (BQ, 16))
    tc_ref[...] = jnp.broadcast_to(tcol, (BQ, 16))


def _sc_gather(d2_table, lab_table, rows, blks):
    mesh = plsc.VectorSubcoreMesh(core_axis_name="c", subcore_axis_name="s")

    @functools.partial(
        pl.kernel,
        mesh=mesh,
        out_type=(
            jax.ShapeDtypeStruct((R, SUB), jnp.float32),
            jax.ShapeDtypeStruct((R, SUB), jnp.int32),
        ),
        scratch_types=[
            [pltpu.VMEM((CH,), jnp.int32) for _ in range(NBUF)],
            [pltpu.VMEM((CH,), jnp.int32) for _ in range(NBUF)],
            [pltpu.VMEM((CH, SUB), jnp.float32) for _ in range(NBUF)],
            [pltpu.VMEM((CH, SUB), jnp.int32) for _ in range(NBUF)],
            [pltpu.SemaphoreType.DMA for _ in range(6 * NBUF)],
        ],
    )
    def gather(d2_hbm, lab_hbm, rows_hbm, blks_hbm, cand_hbm, clab_hbm,
               ridx_v, bidx_v, rows_v, labs_v, sems):
        wid = lax.axis_index("s") * 2 + lax.axis_index("c")
        base = wid * RPW
        gcp = [None] * NBUF
        ocp = [None] * NBUF

        def fire(ch):
            b = ch % NBUF
            off = base + ch * CH
            i1 = pltpu.async_copy(rows_hbm.at[pl.ds(off, CH)], ridx_v[b],
                                  sems[6 * b])
            i2 = pltpu.async_copy(blks_hbm.at[pl.ds(off, CH)], bidx_v[b],
                                  sems[6 * b + 1])
            i1.wait()
            i2.wait()
            g1 = pltpu.async_copy(d2_hbm.at[ridx_v[b]], rows_v[b],
                                  sems[6 * b + 2])
            g2 = pltpu.async_copy(lab_hbm.at[bidx_v[b]], labs_v[b],
                                  sems[6 * b + 3])
            gcp[b] = (g1, g2)

        for ch in range(min(NBUF, NCHUNK)):
            fire(ch)
        for ch in range(NCHUNK):
            b = ch % NBUF
            off = base + ch * CH
            gcp[b][0].wait()
            gcp[b][1].wait()
            o1 = pltpu.async_copy(rows_v[b], cand_hbm.at[pl.ds(off, CH)],
                                  sems[6 * b + 4])
            o2 = pltpu.async_copy(labs_v[b], clab_hbm.at[pl.ds(off, CH)],
                                  sems[6 * b + 5])
            ocp[b] = (o1, o2)
            if ch + NBUF < NCHUNK:
                ocp[b][0].wait()
                ocp[b][1].wait()
                fire(ch + NBUF)
        for b in range(min(NBUF, NCHUNK)):
            ocp[b][0].wait()
            ocp[b][1].wait()

    return gather(d2_table, lab_table, rows, blks)


QPW = NQ // NWORK
NCLS_PAD = 1008


VB = 4
NVR = QPW // VB


def _sc_vote(cand_f, clab_f, trep_f, tcrep_f):
    mesh = plsc.VectorSubcoreMesh(core_axis_name="c", subcore_axis_name="s")

    @functools.partial(
        pl.kernel,
        mesh=mesh,
        out_type=jax.ShapeDtypeStruct((NQ * NCLS,), jnp.float32),
        compiler_params=pltpu.CompilerParams(
            needs_layout_passes=False, use_tc_tiling_on_sc=False),
        scratch_types=[
            [pltpu.VMEM((VB * CAND,), jnp.float32) for _ in range(2)],
            [pltpu.VMEM((VB * CAND,), jnp.int32) for _ in range(2)],
            pltpu.VMEM((QPW * 16,), jnp.int32),
            pltpu.VMEM((QPW * 16,), jnp.int32),
            [pltpu.VMEM((VB * NCLS,), jnp.float32) for _ in range(2)],
            [pltpu.SemaphoreType.DMA for _ in range(6)],
        ],
    )
    def vote(cand_hbm, clab_hbm, t_hbm, tc_hbm, probs_hbm,
             cbuf, lbuf, tall_v, tcall_v, counts, sems):
        wid = lax.axis_index("s") * 2 + lax.axis_index("c")
        base_q = wid * QPW
        lanes = lax.broadcasted_iota(jnp.int32, (16,), 0)
        ones = jnp.ones((16,), jnp.float32)
        zeros = jnp.zeros((16,), jnp.float32)

        pltpu.sync_copy(t_hbm.at[pl.ds(base_q * 16, QPW * 16)], tall_v)
        pltpu.sync_copy(tc_hbm.at[pl.ds(base_q * 16, QPW * 16)], tcall_v)

        lc = [None, None]
        ll = [None, None]
        ocp = [None, None]

        def fire(r):
            b = r % 2
            off = (base_q + r * VB) * CAND
            lc[b] = pltpu.async_copy(cand_hbm.at[pl.ds(off, VB * CAND)],
                                     cbuf[b], sems[2 * b])
            ll[b] = pltpu.async_copy(clab_hbm.at[pl.ds(off, VB * CAND)],
                                     lbuf[b], sems[2 * b + 1])

        fire(0)
        fire(1)
        for r in range(NVR):
            b = r % 2
            lc[b].wait()
            ll[b].wait()
            if r >= 2:
                ocp[b].wait()

            def zbody(k, _):
                counts[b][pl.ds(k * 16, 16)] = zeros
                return 0

            lax.fori_loop(0, VB * NCLS // 16, zbody, 0)

            for qi in range(VB):
                qrow = r * VB + qi
                tv = tall_v[pl.ds(qrow * 16, 16)]
                tcv = tcall_v[pl.ds(qrow * 16, 16)]

                def cbody(i, _):
                    for u in range(8):
                        off0 = i * 128 + u * 16
                        cs = _sortable(cbuf[b][pl.ds(qi * CAND + off0, 16)])
                        lv = lbuf[b][pl.ds(qi * CAND + off0, 16)]
                        col = lanes + off0
                        sel = (cs < tv) | ((cs == tv) & (col <= tcv))
                        plsc.addupdate_scatter(
                            counts[b], [lv + qi * NCLS], ones, mask=sel)
                    return 0

                lax.fori_loop(0, CAND // 128, cbody, 0)

            def dbody(k, _):
                sl = pl.ds(k * 16, 16)
                counts[b][sl] = counts[b][sl] / 33.0
                return 0

            lax.fori_loop(0, VB * NCLS // 16, dbody, 0)
            ocp[b] = pltpu.async_copy(
                counts[b],
                probs_hbm.at[pl.ds((base_q + r * VB) * NCLS, VB * NCLS)],
                sems[4 + b])
            if r + 2 < NVR:
                fire(r + 2)
        ocp[0].wait()
        ocp[1].wait()

    return vote(cand_f, clab_f, trep_f, tcrep_f)


def kernel(Z_image, keys, labels):
    kt = jnp.pad(keys, ((0, KPAD - K), (0, 0))).T
    lab_table = jnp.pad(labels, (0, KPAD - K)).reshape(NB, SUB)

    d2, bm3 = pl.pallas_call(
        _dist_kernel,
        grid=(KPAD // BK,),
        in_specs=[
            pl.BlockSpec((NQ, D), lambda j: (0, 0)),
            pl.BlockSpec((D, BK), lambda j: (0, j)),
        ],
        out_specs=[
            pl.BlockSpec((NQ, BK), lambda j: (0, j)),
            pl.BlockSpec((1, 1, NQ, BK // SUB), lambda j: (0, j, 0, 0)),
        ],
        out_shape=[
            jax.ShapeDtypeStruct((NQ, KPAD), jnp.float32),
            jax.ShapeDtypeStruct(
                (1, KPAD // BK, NQ, BK // SUB), jnp.float32),
        ],
    )(Z_image, kt)
    bm = bm3.reshape(KPAD // BK, NQ, BK // SUB).transpose(1, 0, 2).reshape(NQ, NB)

    rows, blks, bmv = pl.pallas_call(
        _select_kernel,
        in_specs=[pl.BlockSpec((NQ, NB), lambda: (0, 0))],
        out_specs=[
            pl.BlockSpec((NQ, NN), lambda: (0, 0)),
            pl.BlockSpec((NQ, NN), lambda: (0, 0)),
            pl.BlockSpec((NQ, NN), lambda: (0, 0)),
        ],
        out_shape=[
            jax.ShapeDtypeStruct((NQ, NN), jnp.int32),
            jax.ShapeDtypeStruct((NQ, NN), jnp.int32),
            jax.ShapeDtypeStruct((NQ, NN), jnp.float32),
        ],
    )(bm)

    cand, clab = _sc_gather(
        d2.reshape(NQ * NB, SUB), lab_table,
        rows.reshape(R), blks.reshape(R))

    cand2 = cand.reshape(NQ, CAND)
    clab2 = clab.reshape(NQ, CAND)

    trep, tcrep = pl.pallas_call(
        _thresh_kernel,
        grid=(NQ // BQ,),
        in_specs=[
            pl.BlockSpec((BQ, CAND), lambda i: (i, 0)),
            pl.BlockSpec((BQ, NN), lambda i: (i, 0)),
        ],
        out_specs=[
            pl.BlockSpec((BQ, 16), lambda i: (i, 0)),
            pl.BlockSpec((BQ, 16), lambda i: (i, 0)),
        ],
        out_shape=[
            jax.ShapeDtypeStruct((NQ, 16), jnp.int32),
            jax.ShapeDtypeStruct((NQ, 16), jnp.int32),
        ],
    )(cand2, bmv)

    probs_flat = _sc_vote(
        cand.reshape(R * SUB), clab.reshape(R * SUB),
        trep.reshape(NQ * 16), tcrep.reshape(NQ * 16))
    return probs_flat.reshape(NQ, NCLS)
